# baseline (speedup 1.0000x reference)
import functools

import jax
import jax.numpy as jnp
from jax import lax
from jax.experimental import pallas as pl
from jax.experimental.pallas import tpu as pltpu
from jax.experimental.pallas import tpu_sc as plsc

_VOCAB = 1000
_VP = 1024
_D = 64
_B, _T = 4, 4096
_N = _B * _T
_NC = 2
_NS = 16
_NW = _NC * _NS
_TPT = _N // _NW
_WPB = _T // _TPT
_CH = 128
_NCH = _TPT // _CH
_RPT = _VP // _NS


def _phase1_body(x_hbm, tok_hbm, out_a, out_cnt,
                 idx_v, x_v, xs_v, ones_v, z128, z16,
                 acc_a, acc_cnt,
                 sem_in, sem_z, sem_s, sem_out):
    c = lax.axis_index("c")
    s = lax.axis_index("s")
    w = s * _NC + c
    b = w // _WPB
    t0 = (w % _WPB) * _TPT

    ld_idx = pltpu.async_copy(tok_hbm.at[w], idx_v, sem_in)
    ld_x = pltpu.async_copy(x_hbm.at[b, pl.ds(t0, _TPT)], x_v, sem_in)

    zeros = jnp.zeros((16,), jnp.float32)
    ones = jnp.ones((16,), jnp.float32)

    def zrow(r, carry):
        for j in range(8):
            z128[r, pl.ds(j * 16, 16)] = zeros
        z16[r] = zeros
        return carry
    lax.fori_loop(0, _RPT, zrow, 0)

    def onesrow(r, carry):
        ones_v[r] = ones
        return carry
    lax.fori_loop(0, _CH, onesrow, 0)

    rows = pl.ds(s * _RPT, _RPT)
    z1 = pltpu.async_copy(z128, acc_a.at[rows], sem_z)
    z2 = pltpu.async_copy(z16, acc_cnt.at[rows], sem_z)

    ld_idx.wait()
    ld_x.wait()

    def sqrow(r, carry):
        for j in range(4):
            v = x_v[r, pl.ds(j * 16, 16)]
            xs_v[r, pl.ds(j * 16, 16)] = v
            xs_v[r, pl.ds(_D + j * 16, 16)] = v * v
        return carry
    lax.fori_loop(0, _TPT, sqrow, 0)

    z1.wait()
    z2.wait()
    plsc.subcore_barrier()

    cps = []
    for ch in range(_NCH):
        idx = idx_v.at[ch]
        sl = pl.ds(ch * _CH, _CH)
        cps.append(pltpu.async_copy(xs_v.at[sl], acc_a.at[idx], sem_s, add=True))
        cps.append(pltpu.async_copy(ones_v, acc_cnt.at[idx], sem_s, add=True))
    for cp in cps:
        cp.wait()

    plsc.subcore_barrier()

    o1 = pltpu.async_copy(acc_a.at[rows], out_a.at[c, pl.ds(s * _RPT, _RPT)],
                          sem_out)
    o2 = pltpu.async_copy(acc_cnt.at[rows], out_cnt.at[c, pl.ds(s * _RPT, _RPT)],
                          sem_out)
    o1.wait()
    o2.wait()


_phase1 = functools.partial(
    pl.kernel,
    out_type=(
        jax.ShapeDtypeStruct((_NC, _VP, 2 * _D), jnp.float32),
        jax.ShapeDtypeStruct((_NC, _VP, 16), jnp.float32),
    ),
    mesh=plsc.VectorSubcoreMesh(
        core_axis_name="c", subcore_axis_name="s",
        num_cores=_NC, num_subcores=_NS),
    scratch_types=[
        pltpu.VMEM((_NCH, _CH), jnp.int32),
        pltpu.VMEM((_TPT, _D), jnp.float32),
        pltpu.VMEM((_TPT, 2 * _D), jnp.float32),
        pltpu.VMEM((_CH, 16), jnp.float32),
        pltpu.VMEM((_RPT, 2 * _D), jnp.float32),
        pltpu.VMEM((_RPT, 16), jnp.float32),
        pltpu.VMEM_SHARED((_VP, 2 * _D), jnp.float32),
        pltpu.VMEM_SHARED((_VP, 16), jnp.float32),
        pltpu.SemaphoreType.DMA,
        pltpu.SemaphoreType.DMA,
        pltpu.SemaphoreType.DMA,
        pltpu.SemaphoreType.DMA,
    ],
    compiler_params=pltpu.CompilerParams(use_tc_tiling_on_sc=False),
)(_phase1_body)


def _finalize_body(a_ref, cnt_ref, loss_ref, nrep_ref):
    a = a_ref[0] + a_ref[1]
    sums = a[:, 0:_D]
    sqs = a[:, _D:2 * _D]
    cnt = cnt_ref[0] + cnt_ref[1]
    c = cnt[:, 0:1]
    mean = sums / jnp.maximum(c, 1.0)
    ss = sqs - c * mean * mean
    var = ss / jnp.maximum(c - 1.0, 1.0)
    var_mean = jnp.sum(var, axis=1, keepdims=True) * (1.0 / _D)
    repeated = c >= 2.0
    nrep = jnp.sum(repeated.astype(jnp.float32))
    total = jnp.sum(jnp.where(repeated, var_mean, 0.0))
    avg = total / jnp.maximum(nrep, 1.0)
    loss = jnp.maximum(1.0 - avg, 0.0)
    loss = jnp.where(nrep > 0.0, loss, 0.0)
    loss_ref[0, 0] = loss
    nrep_ref[0, 0] = nrep.astype(jnp.int32)


_finalize = pl.pallas_call(
    _finalize_body,
    out_shape=(
        jax.ShapeDtypeStruct((1, 1), jnp.float32),
        jax.ShapeDtypeStruct((1, 1), jnp.int32),
    ),
    out_specs=(
        pl.BlockSpec(memory_space=pltpu.SMEM),
        pl.BlockSpec(memory_space=pltpu.SMEM),
    ),
)


@jax.jit
def kernel(semantic_state, token_ids):
    tok = token_ids.reshape(_NW, _NCH, _CH).astype(jnp.int32)
    pa, pcnt = _phase1(semantic_state, tok)
    loss, nrep = _finalize(pa, pcnt)
    return loss[0, 0], nrep[0, 0]


# speedup vs baseline: 1.2381x; 1.2381x over previous
import functools

import jax
import jax.numpy as jnp
from jax import lax
from jax.experimental import pallas as pl
from jax.experimental.pallas import tpu as pltpu
from jax.experimental.pallas import tpu_sc as plsc

_VOCAB = 1000
_VP = 1024
_D = 64
_B, _T = 4, 4096
_N = _B * _T
_NC = 2
_NS = 16
_NW = _NC * _NS
_TPT = _N // _NW
_WPB = _T // _TPT
_CH = 128
_NCH = _TPT // _CH
_RPT = _VP // _NS


def _phase1_body(x_hbm, tok_hbm, out_a, out_cnt,
                 idx_v, x_v, sq_v, ones_v, z64, z16,
                 acc_sum, acc_sq, acc_cnt,
                 sem_in, sem_z, sem_s, sem_out):
    c = lax.axis_index("c")
    s = lax.axis_index("s")
    w = s * _NC + c
    b = w // _WPB
    t0 = (w % _WPB) * _TPT

    ld_idx = pltpu.async_copy(tok_hbm.at[w], idx_v, sem_in)
    ld_x = pltpu.async_copy(x_hbm.at[b, pl.ds(t0, _TPT)], x_v, sem_in)

    zeros = jnp.zeros((16,), jnp.float32)
    ones = jnp.ones((16,), jnp.float32)

    def zrow(r, carry):
        for j in range(4):
            z64[r, pl.ds(j * 16, 16)] = zeros
        z16[r] = zeros
        return carry
    lax.fori_loop(0, _RPT, zrow, 0)

    def onesrow(r, carry):
        ones_v[r] = ones
        return carry
    lax.fori_loop(0, _CH, onesrow, 0)

    rows = pl.ds(s * _RPT, _RPT)
    z1 = pltpu.async_copy(z64, acc_sum.at[rows], sem_z)
    z2 = pltpu.async_copy(z64, acc_sq.at[rows], sem_z)
    z3 = pltpu.async_copy(z16, acc_cnt.at[rows], sem_z)

    ld_idx.wait()
    ld_x.wait()

    def sqrow(r, carry):
        for j in range(4):
            v = x_v[r, pl.ds(j * 16, 16)]
            sq_v[r, pl.ds(j * 16, 16)] = v * v
        return carry
    lax.fori_loop(0, _TPT, sqrow, 0)

    z1.wait()
    z2.wait()
    z3.wait()
    plsc.subcore_barrier()

    cps = []
    for ch in range(_NCH):
        idx = idx_v.at[ch]
        sl = pl.ds(ch * _CH, _CH)
        cps.append(pltpu.async_copy(x_v.at[sl], acc_sum.at[idx], sem_s, add=True))
        cps.append(pltpu.async_copy(sq_v.at[sl], acc_sq.at[idx], sem_s, add=True))
        cps.append(pltpu.async_copy(ones_v, acc_cnt.at[idx], sem_s, add=True))
    for cp in cps:
        cp.wait()

    plsc.subcore_barrier()

    o1 = pltpu.async_copy(acc_sum.at[rows], out_a.at[c, rows, pl.ds(0, _D)],
                          sem_out)
    o2 = pltpu.async_copy(acc_sq.at[rows], out_a.at[c, rows, pl.ds(_D, _D)],
                          sem_out)
    o3 = pltpu.async_copy(acc_cnt.at[rows], out_cnt.at[c, rows], sem_out)
    o1.wait()
    o2.wait()
    o3.wait()


_phase1 = functools.partial(
    pl.kernel,
    out_type=(
        jax.ShapeDtypeStruct((_NC, _VP, 2 * _D), jnp.float32),
        jax.ShapeDtypeStruct((_NC, _VP, 16), jnp.float32),
    ),
    mesh=plsc.VectorSubcoreMesh(
        core_axis_name="c", subcore_axis_name="s",
        num_cores=_NC, num_subcores=_NS),
    scratch_types=[
        pltpu.VMEM((_NCH, _CH), jnp.int32),
        pltpu.VMEM((_TPT, _D), jnp.float32),
        pltpu.VMEM((_TPT, _D), jnp.float32),
        pltpu.VMEM((_CH, 16), jnp.float32),
        pltpu.VMEM((_RPT, _D), jnp.float32),
        pltpu.VMEM((_RPT, 16), jnp.float32),
        pltpu.VMEM_SHARED((_VP, _D), jnp.float32),
        pltpu.VMEM_SHARED((_VP, _D), jnp.float32),
        pltpu.VMEM_SHARED((_VP, 16), jnp.float32),
        pltpu.SemaphoreType.DMA,
        pltpu.SemaphoreType.DMA,
        pltpu.SemaphoreType.DMA,
        pltpu.SemaphoreType.DMA,
    ],
    compiler_params=pltpu.CompilerParams(use_tc_tiling_on_sc=False),
)(_phase1_body)


def _finalize_body(a_ref, cnt_ref, loss_ref, nrep_ref):
    a = a_ref[0] + a_ref[1]
    cnt = cnt_ref[0] + cnt_ref[1]
    c = cnt[:, 0:1]
    cm = jnp.maximum(c, 1.0)
    lane = lax.broadcasted_iota(jnp.int32, (_VP, 2 * _D), 1)
    contrib = jnp.where(lane >= _D, a, -(a * a) / cm)
    ss_sum = jnp.sum(contrib, axis=1, keepdims=True)
    var_mean = ss_sum / (jnp.maximum(c - 1.0, 1.0) * _D)
    repeated = c >= 2.0
    nrep = jnp.sum(repeated.astype(jnp.float32))
    total = jnp.sum(jnp.where(repeated, var_mean, 0.0))
    avg = total / jnp.maximum(nrep, 1.0)
    loss = jnp.maximum(1.0 - avg, 0.0)
    loss = jnp.where(nrep > 0.0, loss, 0.0)
    loss_ref[0, 0] = loss
    nrep_ref[0, 0] = nrep.astype(jnp.int32)


_finalize = pl.pallas_call(
    _finalize_body,
    out_shape=(
        jax.ShapeDtypeStruct((1, 1), jnp.float32),
        jax.ShapeDtypeStruct((1, 1), jnp.int32),
    ),
    out_specs=(
        pl.BlockSpec(memory_space=pltpu.SMEM),
        pl.BlockSpec(memory_space=pltpu.SMEM),
    ),
)


@jax.jit
def kernel(semantic_state, token_ids):
    tok = token_ids.reshape(_NW, _NCH, _CH).astype(jnp.int32)
    pa, pcnt = _phase1(semantic_state, tok)
    loss, nrep = _finalize(pa, pcnt)
    return loss[0, 0], nrep[0, 0]
